# baseline (speedup 1.0000x reference)
import functools

import jax
import jax.numpy as jnp
from jax.experimental import pallas as pl
from jax.experimental.pallas import tpu as pltpu
from jax.experimental.pallas import tpu_sc as plsc

T = 2048
C = 768
H = 4 * C
E = 16
BLK = 128
NBLK = T // BLK + E
NW = 32
TPW = T // NW


def _router_body(x_ref, wg_ref, pos_ref, map_ref):
    x = x_ref[...]
    wg = wg_ref[...]
    logits = jax.lax.dot_general(
        x, wg, (((1,), (1,)), ((), ())), preferred_element_type=jnp.float32)
    lane = jax.lax.broadcasted_iota(jnp.int32, (T, E), 1)
    m = jnp.max(logits, axis=1, keepdims=True)
    eid = jnp.min(jnp.where(logits == m, lane, E), axis=1, keepdims=True)
    onehot = (lane == eid).astype(jnp.float32)

    tri = (jax.lax.broadcasted_iota(jnp.int32, (T, T), 0)
           > jax.lax.broadcasted_iota(jnp.int32, (T, T), 1)).astype(jnp.float32)
    rank_mat = jax.lax.dot_general(
        tri, onehot, (((1,), (0,)), ((), ())), preferred_element_type=jnp.float32)
    rank = jnp.sum(rank_mat * onehot, axis=1, keepdims=True)

    counts = jnp.sum(onehot, axis=0, keepdims=True)
    nb = jnp.floor((counts + float(BLK - 1)) * (1.0 / BLK))
    erow = jax.lax.broadcasted_iota(jnp.int32, (E, E), 0)
    ecol = jax.lax.broadcasted_iota(jnp.int32, (E, E), 1)
    upper = (erow < ecol).astype(jnp.float32)
    bstart = jax.lax.dot_general(
        nb, upper, (((1,), (0,)), ((), ())), preferred_element_type=jnp.float32)

    posf = float(BLK) * jnp.sum(onehot * bstart, axis=1, keepdims=True) + rank
    pos_ref[...] = posf.astype(jnp.int32)

    ones_col = jnp.ones((T, 1), jnp.float32)
    counts_col = jax.lax.dot_general(
        onehot, ones_col, (((0,), (0,)), ((), ())), preferred_element_type=jnp.float32)
    nb_col = jnp.floor((counts_col + float(BLK - 1)) * (1.0 / BLK))
    lower = (ecol < erow).astype(jnp.float32)
    bstart_col = jax.lax.dot_general(
        lower, nb_col, (((1,), (0,)), ((), ())), preferred_element_type=jnp.float32)
    map_ref[0:E, :] = bstart_col.astype(jnp.int32)
    map_ref[E:2 * E, :] = nb_col.astype(jnp.int32)


_router = pl.pallas_call(
    _router_body,
    out_shape=(
        jax.ShapeDtypeStruct((T, 1), jnp.int32),
        jax.ShapeDtypeStruct((2 * E, 1), jnp.int32),
    ),
)


NSPLIT = 4
H2 = H // NSPLIT


def _ffn_body(m_ref, d_ref, w1_ref, w2_ref, y_ref):
    e = pl.program_id(0)
    j = pl.program_id(1)
    bs = m_ref[e]
    nb = m_ref[E + e]
    def blk(i, carry):
        r0 = (bs + i) * BLK
        xb = d_ref[pl.ds(r0, BLK), :]
        h = jax.lax.dot_general(
            xb, w1_ref[0], (((1,), (1,)), ((), ())),
            preferred_element_type=jnp.float32)
        h = 0.5 * h * (1.0 + jax.lax.erf(h * (2.0 ** -0.5)))
        part = jax.lax.dot_general(
            h, w2_ref[0], (((1,), (1,)), ((), ())),
            preferred_element_type=jnp.float32)

        @pl.when(j == 0)
        def _():
            y_ref[pl.ds(r0, BLK), :] = part

        @pl.when(j != 0)
        def _():
            y_ref[pl.ds(r0, BLK), :] += part

        return carry

    jax.lax.fori_loop(0, nb, blk, 0)


_ffn = pl.pallas_call(
    _ffn_body,
    grid_spec=pltpu.PrefetchScalarGridSpec(
        num_scalar_prefetch=1,
        grid=(E, NSPLIT),
        in_specs=[
            pl.BlockSpec((NBLK * BLK, C), lambda e, j, m: (0, 0)),
            pl.BlockSpec((1, H2, C), lambda e, j, m: (e, j, 0)),
            pl.BlockSpec((1, C, H2), lambda e, j, m: (e, 0, j)),
        ],
        out_specs=pl.BlockSpec((NBLK * BLK, C), lambda e, j, m: (0, 0)),
    ),
    out_shape=jax.ShapeDtypeStruct((NBLK * BLK, C), jnp.float32),
)


@functools.lru_cache(maxsize=None)
def _sc_kernels():
    mesh = plsc.VectorSubcoreMesh(core_axis_name="c", subcore_axis_name="s")
    scratch = [
        pltpu.VMEM((TPW,), jnp.int32),
        pltpu.VMEM((TPW, C), jnp.float32),
        pltpu.SemaphoreType.DMA,
    ]

    @functools.partial(
        pl.kernel,
        out_type=jax.ShapeDtypeStruct((NBLK * BLK, C), jnp.float32),
        mesh=mesh,
        scratch_types=scratch,
    )
    def dispatch_sc(x_hbm, pos_hbm, out_hbm, idx_v, rows_v, sem):
        wid = jax.lax.axis_index("s") * 2 + jax.lax.axis_index("c")
        base = wid * TPW
        pltpu.sync_copy(pos_hbm.at[pl.ds(base, TPW)], idx_v)
        pltpu.sync_copy(x_hbm.at[pl.ds(base, TPW)], rows_v)
        pltpu.async_copy(rows_v, out_hbm.at[idx_v], sem).wait()

    @functools.partial(
        pl.kernel,
        out_type=jax.ShapeDtypeStruct((T, C), jnp.float32),
        mesh=mesh,
        scratch_types=scratch,
    )
    def combine_sc(y_hbm, pos_hbm, out_hbm, idx_v, rows_v, sem):
        wid = jax.lax.axis_index("s") * 2 + jax.lax.axis_index("c")
        base = wid * TPW
        pltpu.sync_copy(pos_hbm.at[pl.ds(base, TPW)], idx_v)
        pltpu.async_copy(y_hbm.at[idx_v], rows_v, sem).wait()
        pltpu.sync_copy(rows_v, out_hbm.at[pl.ds(base, TPW)])

    return dispatch_sc, combine_sc


def kernel(x, Wg, W1, W2):
    B, T_, C_ = x.shape
    xf = x.reshape(T_, C_)
    pos2d, map2d = _router(xf, Wg)
    pos = pos2d.reshape(T_)
    blk_map = map2d.reshape(2 * E)
    dispatch_sc, combine_sc = _sc_kernels()
    disp = dispatch_sc(xf, pos)
    y = _ffn(blk_map, disp, W1, W2)
    out = combine_sc(y, pos)
    return out.reshape(B, T_, C_)


# speedup vs baseline: 1.0201x; 1.0201x over previous
import functools

import jax
import jax.numpy as jnp
from jax.experimental import pallas as pl
from jax.experimental.pallas import tpu as pltpu
from jax.experimental.pallas import tpu_sc as plsc

T = 2048
C = 768
H = 4 * C
E = 16
BLK = 64
NBLK = T // BLK + E
NW = 32
TPW = T // NW


def _router_body(x_ref, wg_ref, pos_ref, map_ref):
    x = x_ref[...]
    wg = wg_ref[...]
    logits = jax.lax.dot_general(
        x, wg, (((1,), (1,)), ((), ())), preferred_element_type=jnp.float32)
    lane = jax.lax.broadcasted_iota(jnp.int32, (T, E), 1)
    m = jnp.max(logits, axis=1, keepdims=True)
    eid = jnp.min(jnp.where(logits == m, lane, E), axis=1, keepdims=True)
    onehot = (lane == eid).astype(jnp.float32)

    tri = (jax.lax.broadcasted_iota(jnp.int32, (T, T), 0)
           > jax.lax.broadcasted_iota(jnp.int32, (T, T), 1)).astype(jnp.float32)
    rank_mat = jax.lax.dot_general(
        tri, onehot, (((1,), (0,)), ((), ())), preferred_element_type=jnp.float32)
    rank = jnp.sum(rank_mat * onehot, axis=1, keepdims=True)

    counts = jnp.sum(onehot, axis=0, keepdims=True)
    nb = jnp.floor((counts + float(BLK - 1)) * (1.0 / BLK))
    erow = jax.lax.broadcasted_iota(jnp.int32, (E, E), 0)
    ecol = jax.lax.broadcasted_iota(jnp.int32, (E, E), 1)
    upper = (erow < ecol).astype(jnp.float32)
    bstart = jax.lax.dot_general(
        nb, upper, (((1,), (0,)), ((), ())), preferred_element_type=jnp.float32)

    posf = float(BLK) * jnp.sum(onehot * bstart, axis=1, keepdims=True) + rank
    pos_ref[...] = posf.astype(jnp.int32)

    ones_col = jnp.ones((T, 1), jnp.float32)
    counts_col = jax.lax.dot_general(
        onehot, ones_col, (((0,), (0,)), ((), ())), preferred_element_type=jnp.float32)
    nb_col = jnp.floor((counts_col + float(BLK - 1)) * (1.0 / BLK))
    lower = (ecol < erow).astype(jnp.float32)
    bstart_col = jax.lax.dot_general(
        lower, nb_col, (((1,), (0,)), ((), ())), preferred_element_type=jnp.float32)
    map_ref[0:E, :] = bstart_col.astype(jnp.int32)
    map_ref[E:2 * E, :] = nb_col.astype(jnp.int32)


_router = pl.pallas_call(
    _router_body,
    out_shape=(
        jax.ShapeDtypeStruct((T, 1), jnp.int32),
        jax.ShapeDtypeStruct((2 * E, 1), jnp.int32),
    ),
)


NSPLIT = 1
H2 = H // NSPLIT


def _ffn_body(m_ref, d_ref, w1_ref, w2_ref, y_ref):
    e = pl.program_id(0)
    j = pl.program_id(1)
    bs = m_ref[e]
    nb = m_ref[E + e]
    def blk(i, carry):
        r0 = (bs + i) * BLK
        xb = d_ref[pl.ds(r0, BLK), :]
        h = jax.lax.dot_general(
            xb, w1_ref[0], (((1,), (1,)), ((), ())),
            preferred_element_type=jnp.float32)
        h = 0.5 * h * (1.0 + jax.lax.erf(h * (2.0 ** -0.5)))
        part = jax.lax.dot_general(
            h, w2_ref[0], (((1,), (1,)), ((), ())),
            preferred_element_type=jnp.float32)

        @pl.when(j == 0)
        def _():
            y_ref[pl.ds(r0, BLK), :] = part

        @pl.when(j != 0)
        def _():
            y_ref[pl.ds(r0, BLK), :] += part

        return carry

    jax.lax.fori_loop(0, nb, blk, 0)


_ffn = pl.pallas_call(
    _ffn_body,
    grid_spec=pltpu.PrefetchScalarGridSpec(
        num_scalar_prefetch=1,
        grid=(E, NSPLIT),
        in_specs=[
            pl.BlockSpec((NBLK * BLK, C), lambda e, j, m: (0, 0)),
            pl.BlockSpec((1, H2, C), lambda e, j, m: (e, j, 0)),
            pl.BlockSpec((1, C, H2), lambda e, j, m: (e, 0, j)),
        ],
        out_specs=pl.BlockSpec((NBLK * BLK, C), lambda e, j, m: (0, 0)),
    ),
    out_shape=jax.ShapeDtypeStruct((NBLK * BLK, C), jnp.float32),
)


@functools.lru_cache(maxsize=None)
def _sc_kernels():
    mesh = plsc.VectorSubcoreMesh(core_axis_name="c", subcore_axis_name="s")
    scratch = [
        pltpu.VMEM((TPW,), jnp.int32),
        pltpu.VMEM((TPW, C), jnp.float32),
        pltpu.SemaphoreType.DMA,
    ]

    @functools.partial(
        pl.kernel,
        out_type=jax.ShapeDtypeStruct((NBLK * BLK, C), jnp.float32),
        mesh=mesh,
        scratch_types=scratch,
    )
    def dispatch_sc(x_hbm, pos_hbm, out_hbm, idx_v, rows_v, sem):
        wid = jax.lax.axis_index("s") * 2 + jax.lax.axis_index("c")
        base = wid * TPW
        pltpu.sync_copy(pos_hbm.at[pl.ds(base, TPW)], idx_v)
        pltpu.sync_copy(x_hbm.at[pl.ds(base, TPW)], rows_v)
        pltpu.async_copy(rows_v, out_hbm.at[idx_v], sem).wait()

    @functools.partial(
        pl.kernel,
        out_type=jax.ShapeDtypeStruct((T, C), jnp.float32),
        mesh=mesh,
        scratch_types=scratch,
    )
    def combine_sc(y_hbm, pos_hbm, out_hbm, idx_v, rows_v, sem):
        wid = jax.lax.axis_index("s") * 2 + jax.lax.axis_index("c")
        base = wid * TPW
        pltpu.sync_copy(pos_hbm.at[pl.ds(base, TPW)], idx_v)
        pltpu.async_copy(y_hbm.at[idx_v], rows_v, sem).wait()
        pltpu.sync_copy(rows_v, out_hbm.at[pl.ds(base, TPW)])

    return dispatch_sc, combine_sc


def kernel(x, Wg, W1, W2):
    B, T_, C_ = x.shape
    xf = x.reshape(T_, C_)
    pos2d, map2d = _router(xf, Wg)
    pos = pos2d.reshape(T_)
    blk_map = map2d.reshape(2 * E)
    dispatch_sc, combine_sc = _sc_kernels()
    disp = dispatch_sc(xf, pos)
    y = _ffn(blk_map, disp, W1, W2)
    out = combine_sc(y, pos)
    return out.reshape(B, T_, C_)


# speedup vs baseline: 1.1231x; 1.1009x over previous
import functools

import jax
import jax.numpy as jnp
from jax.experimental import pallas as pl
from jax.experimental.pallas import tpu as pltpu
from jax.experimental.pallas import tpu_sc as plsc

T = 2048
C = 768
H = 4 * C
E = 16
BLK = 128
NBLK = T // BLK + E
NW = 32
TPW = T // NW


def _router_body(x_ref, wg_ref, pos_ref, map_ref):
    x = x_ref[...]
    wg = wg_ref[...]
    logits = jax.lax.dot_general(
        x, wg, (((1,), (1,)), ((), ())), preferred_element_type=jnp.float32)
    lane = jax.lax.broadcasted_iota(jnp.int32, (T, E), 1)
    m = jnp.max(logits, axis=1, keepdims=True)
    eid = jnp.min(jnp.where(logits == m, lane, E), axis=1, keepdims=True)
    onehot = (lane == eid).astype(jnp.float32)

    tri = (jax.lax.broadcasted_iota(jnp.int32, (T, T), 0)
           > jax.lax.broadcasted_iota(jnp.int32, (T, T), 1)).astype(jnp.float32)
    rank_mat = jax.lax.dot_general(
        tri, onehot, (((1,), (0,)), ((), ())), preferred_element_type=jnp.float32)
    rank = jnp.sum(rank_mat * onehot, axis=1, keepdims=True)

    counts = jnp.sum(onehot, axis=0, keepdims=True)
    nb = jnp.floor((counts + float(BLK - 1)) * (1.0 / BLK))
    erow = jax.lax.broadcasted_iota(jnp.int32, (E, E), 0)
    ecol = jax.lax.broadcasted_iota(jnp.int32, (E, E), 1)
    upper = (erow < ecol).astype(jnp.float32)
    bstart = jax.lax.dot_general(
        nb, upper, (((1,), (0,)), ((), ())), preferred_element_type=jnp.float32)

    posf = float(BLK) * jnp.sum(onehot * bstart, axis=1, keepdims=True) + rank
    pos_ref[...] = posf.astype(jnp.int32)

    ones_col = jnp.ones((T, 1), jnp.float32)
    counts_col = jax.lax.dot_general(
        onehot, ones_col, (((0,), (0,)), ((), ())), preferred_element_type=jnp.float32)
    nb_col = jnp.floor((counts_col + float(BLK - 1)) * (1.0 / BLK))
    lower = (ecol < erow).astype(jnp.float32)
    bstart_col = jax.lax.dot_general(
        lower, nb_col, (((1,), (0,)), ((), ())), preferred_element_type=jnp.float32)
    map_ref[0:E, :] = bstart_col.astype(jnp.int32)
    map_ref[E:2 * E, :] = nb_col.astype(jnp.int32)


_router = pl.pallas_call(
    _router_body,
    out_shape=(
        jax.ShapeDtypeStruct((T, 1), jnp.int32),
        jax.ShapeDtypeStruct((2 * E, 1), jnp.int32),
    ),
)


NSPLIT = 2
H2 = H // NSPLIT


def _ffn_body(m_ref, d_ref, w1_ref, w2_ref, y_ref):
    e = pl.program_id(0)
    j = pl.program_id(1)
    bs = m_ref[e]
    nb = m_ref[E + e]
    def blk(i, carry):
        r0 = (bs + i) * BLK
        xb = d_ref[pl.ds(r0, BLK), :]
        h = jax.lax.dot_general(
            xb, w1_ref[0], (((1,), (1,)), ((), ())),
            preferred_element_type=jnp.float32)
        h = 0.5 * h * (1.0 + jax.lax.erf(h * (2.0 ** -0.5)))
        part = jax.lax.dot_general(
            h, w2_ref[0], (((1,), (1,)), ((), ())),
            preferred_element_type=jnp.float32)

        @pl.when(j == 0)
        def _():
            y_ref[pl.ds(r0, BLK), :] = part

        @pl.when(j != 0)
        def _():
            y_ref[pl.ds(r0, BLK), :] += part

        return carry

    jax.lax.fori_loop(0, nb, blk, 0)


_ffn = pl.pallas_call(
    _ffn_body,
    grid_spec=pltpu.PrefetchScalarGridSpec(
        num_scalar_prefetch=1,
        grid=(E, NSPLIT),
        in_specs=[
            pl.BlockSpec((NBLK * BLK, C), lambda e, j, m: (0, 0)),
            pl.BlockSpec((1, H2, C), lambda e, j, m: (e, j, 0)),
            pl.BlockSpec((1, C, H2), lambda e, j, m: (e, 0, j)),
        ],
        out_specs=pl.BlockSpec((NBLK * BLK, C), lambda e, j, m: (0, 0)),
    ),
    out_shape=jax.ShapeDtypeStruct((NBLK * BLK, C), jnp.float32),
)


@functools.lru_cache(maxsize=None)
def _sc_kernels():
    mesh = plsc.VectorSubcoreMesh(core_axis_name="c", subcore_axis_name="s")
    scratch = [
        pltpu.VMEM((TPW,), jnp.int32),
        pltpu.VMEM((TPW, C), jnp.float32),
        pltpu.SemaphoreType.DMA,
    ]

    @functools.partial(
        pl.kernel,
        out_type=jax.ShapeDtypeStruct((NBLK * BLK, C), jnp.float32),
        mesh=mesh,
        scratch_types=scratch,
    )
    def dispatch_sc(x_hbm, pos_hbm, out_hbm, idx_v, rows_v, sem):
        wid = jax.lax.axis_index("s") * 2 + jax.lax.axis_index("c")
        base = wid * TPW
        pltpu.sync_copy(pos_hbm.at[pl.ds(base, TPW)], idx_v)
        pltpu.sync_copy(x_hbm.at[pl.ds(base, TPW)], rows_v)
        pltpu.async_copy(rows_v, out_hbm.at[idx_v], sem).wait()

    @functools.partial(
        pl.kernel,
        out_type=jax.ShapeDtypeStruct((T, C), jnp.float32),
        mesh=mesh,
        scratch_types=scratch,
    )
    def combine_sc(y_hbm, pos_hbm, out_hbm, idx_v, rows_v, sem):
        wid = jax.lax.axis_index("s") * 2 + jax.lax.axis_index("c")
        base = wid * TPW
        pltpu.sync_copy(pos_hbm.at[pl.ds(base, TPW)], idx_v)
        pltpu.async_copy(y_hbm.at[idx_v], rows_v, sem).wait()
        pltpu.sync_copy(rows_v, out_hbm.at[pl.ds(base, TPW)])

    return dispatch_sc, combine_sc


def kernel(x, Wg, W1, W2):
    B, T_, C_ = x.shape
    xf = x.reshape(T_, C_)
    pos2d, map2d = _router(xf, Wg)
    pos = pos2d.reshape(T_)
    blk_map = map2d.reshape(2 * E)
    dispatch_sc, combine_sc = _sc_kernels()
    disp = dispatch_sc(xf, pos)
    y = _ffn(blk_map, disp, W1, W2)
    out = combine_sc(y, pos)
    return out.reshape(B, T_, C_)


# speedup vs baseline: 1.1255x; 1.0022x over previous
import functools

import jax
import jax.numpy as jnp
from jax.experimental import pallas as pl
from jax.experimental.pallas import tpu as pltpu
from jax.experimental.pallas import tpu_sc as plsc

T = 2048
C = 768
H = 4 * C
E = 16
BLK = 128
NBLK = T // BLK + E
NW = 32
TPW = T // NW


def _router_body(x_ref, wg_ref, pos_ref, map_ref):
    x = x_ref[...]
    wg = wg_ref[...]
    logits = jax.lax.dot_general(
        x, wg, (((1,), (1,)), ((), ())), preferred_element_type=jnp.float32)
    lane = jax.lax.broadcasted_iota(jnp.int32, (T, E), 1)
    m = jnp.max(logits, axis=1, keepdims=True)
    eid = jnp.min(jnp.where(logits == m, lane, E), axis=1, keepdims=True)
    onehot = (lane == eid).astype(jnp.float32)

    tri = (jax.lax.broadcasted_iota(jnp.int32, (T, T), 0)
           > jax.lax.broadcasted_iota(jnp.int32, (T, T), 1)).astype(jnp.bfloat16)
    rank_mat = jax.lax.dot_general(
        tri, onehot.astype(jnp.bfloat16), (((1,), (0,)), ((), ())),
        preferred_element_type=jnp.float32)
    rank = jnp.sum(rank_mat * onehot, axis=1, keepdims=True)

    counts = jnp.sum(onehot, axis=0, keepdims=True)
    nb = jnp.floor((counts + float(BLK - 1)) * (1.0 / BLK))
    erow = jax.lax.broadcasted_iota(jnp.int32, (E, E), 0)
    ecol = jax.lax.broadcasted_iota(jnp.int32, (E, E), 1)
    upper = (erow < ecol).astype(jnp.float32)
    bstart = jax.lax.dot_general(
        nb, upper, (((1,), (0,)), ((), ())), preferred_element_type=jnp.float32)

    posf = float(BLK) * jnp.sum(onehot * bstart, axis=1, keepdims=True) + rank
    pos_ref[...] = posf.astype(jnp.int32)

    ones_col = jnp.ones((T, 1), jnp.float32)
    counts_col = jax.lax.dot_general(
        onehot, ones_col, (((0,), (0,)), ((), ())), preferred_element_type=jnp.float32)
    nb_col = jnp.floor((counts_col + float(BLK - 1)) * (1.0 / BLK))
    lower = (ecol < erow).astype(jnp.float32)
    bstart_col = jax.lax.dot_general(
        lower, nb_col, (((1,), (0,)), ((), ())), preferred_element_type=jnp.float32)
    map_ref[0:E, :] = bstart_col.astype(jnp.int32)
    map_ref[E:2 * E, :] = nb_col.astype(jnp.int32)


_router = pl.pallas_call(
    _router_body,
    out_shape=(
        jax.ShapeDtypeStruct((T, 1), jnp.int32),
        jax.ShapeDtypeStruct((2 * E, 1), jnp.int32),
    ),
)


NSPLIT = 2
H2 = H // NSPLIT


def _ffn_body(m_ref, d_ref, w1_ref, w2_ref, y_ref):
    e = pl.program_id(0)
    j = pl.program_id(1)
    bs = m_ref[e]
    nb = m_ref[E + e]
    def blk(i, carry):
        r0 = (bs + i) * BLK
        xb = d_ref[pl.ds(r0, BLK), :]
        h = jax.lax.dot_general(
            xb, w1_ref[0], (((1,), (1,)), ((), ())),
            preferred_element_type=jnp.float32)
        h = 0.5 * h * (1.0 + jax.lax.erf(h * (2.0 ** -0.5)))
        part = jax.lax.dot_general(
            h, w2_ref[0], (((1,), (1,)), ((), ())),
            preferred_element_type=jnp.float32)

        @pl.when(j == 0)
        def _():
            y_ref[pl.ds(r0, BLK), :] = part

        @pl.when(j != 0)
        def _():
            y_ref[pl.ds(r0, BLK), :] += part

        return carry

    jax.lax.fori_loop(0, nb, blk, 0)


_ffn = pl.pallas_call(
    _ffn_body,
    grid_spec=pltpu.PrefetchScalarGridSpec(
        num_scalar_prefetch=1,
        grid=(E, NSPLIT),
        in_specs=[
            pl.BlockSpec((NBLK * BLK, C), lambda e, j, m: (0, 0)),
            pl.BlockSpec((1, H2, C), lambda e, j, m: (e, j, 0)),
            pl.BlockSpec((1, C, H2), lambda e, j, m: (e, 0, j)),
        ],
        out_specs=pl.BlockSpec((NBLK * BLK, C), lambda e, j, m: (0, 0)),
    ),
    out_shape=jax.ShapeDtypeStruct((NBLK * BLK, C), jnp.float32),
)


@functools.lru_cache(maxsize=None)
def _sc_kernels():
    mesh = plsc.VectorSubcoreMesh(core_axis_name="c", subcore_axis_name="s")
    scratch = [
        pltpu.VMEM((TPW,), jnp.int32),
        pltpu.VMEM((TPW, C), jnp.float32),
        pltpu.SemaphoreType.DMA,
    ]

    @functools.partial(
        pl.kernel,
        out_type=jax.ShapeDtypeStruct((NBLK * BLK, C), jnp.float32),
        mesh=mesh,
        scratch_types=scratch,
    )
    def dispatch_sc(x_hbm, pos_hbm, out_hbm, idx_v, rows_v, sem):
        wid = jax.lax.axis_index("s") * 2 + jax.lax.axis_index("c")
        base = wid * TPW
        pltpu.sync_copy(pos_hbm.at[pl.ds(base, TPW)], idx_v)
        pltpu.sync_copy(x_hbm.at[pl.ds(base, TPW)], rows_v)
        pltpu.async_copy(rows_v, out_hbm.at[idx_v], sem).wait()

    @functools.partial(
        pl.kernel,
        out_type=jax.ShapeDtypeStruct((T, C), jnp.float32),
        mesh=mesh,
        scratch_types=scratch,
    )
    def combine_sc(y_hbm, pos_hbm, out_hbm, idx_v, rows_v, sem):
        wid = jax.lax.axis_index("s") * 2 + jax.lax.axis_index("c")
        base = wid * TPW
        pltpu.sync_copy(pos_hbm.at[pl.ds(base, TPW)], idx_v)
        pltpu.async_copy(y_hbm.at[idx_v], rows_v, sem).wait()
        pltpu.sync_copy(rows_v, out_hbm.at[pl.ds(base, TPW)])

    return dispatch_sc, combine_sc


def kernel(x, Wg, W1, W2):
    B, T_, C_ = x.shape
    xf = x.reshape(T_, C_)
    pos2d, map2d = _router(xf, Wg)
    pos = pos2d.reshape(T_)
    blk_map = map2d.reshape(2 * E)
    dispatch_sc, combine_sc = _sc_kernels()
    disp = dispatch_sc(xf, pos)
    y = _ffn(blk_map, disp, W1, W2)
    out = combine_sc(y, pos)
    return out.reshape(B, T_, C_)
